# Initial kernel scaffold; baseline (speedup 1.0000x reference)
import jax
import jax.numpy as jnp
from jax.experimental import pallas as pl


def kernel(h, edge_index, W1, b1, W2, b2):
    raise NotImplementedError("write your pallas kernel here")


# speedup vs baseline: 7.5331x; 7.5331x over previous
import functools

import jax
import jax.numpy as jnp
from jax import lax
from jax.experimental import pallas as pl
from jax.experimental.pallas import tpu as pltpu
from jax.experimental.pallas import tpu_sc as plsc

N_NODES = 10000
N_EDGES = 320000
D = 128

NC = 2
NS = 16
NW = NC * NS
EPT = N_EDGES // NW
CHUNK = 80
NCH = EPT // CHUNK
GC = 25
NG = NCH // GC
WCH = 80
NWC = N_NODES // WCH
WPT = -(-NWC // NS)

_mesh = plsc.VectorSubcoreMesh(
    core_axis_name="c", subcore_axis_name="s", num_cores=NC, num_subcores=NS)


def _zero_1d(ref, n):
    def body(i, _):
        ref[pl.ds(i * 16, 16)] = jnp.zeros((16,), jnp.float32)
        return 0
    lax.fori_loop(0, n // 16, body, 0)


def _zero_2d(ref, rows, cols):
    nlc = cols // 16
    def body(i, _):
        ref[i // nlc, pl.ds((i % nlc) * 16, 16)] = jnp.zeros((16,), jnp.float32)
        return 0
    lax.fori_loop(0, rows * nlc, body, 0)


def _deg_body(src_hbm, dst_hbm, oo0, oi0, oo1, oi1,
              idx_s, idx_d, ones_v, zbuf, dego, degi):
    c = lax.axis_index("c")
    s = lax.axis_index("s")
    wid = c * NS + s
    pltpu.sync_copy(src_hbm.at[wid], idx_s)
    pltpu.sync_copy(dst_hbm.at[wid], idx_d)

    def fill_ones(i, _):
        ones_v[pl.ds(i * 16, 16)] = jnp.ones((16,), jnp.float32)
        return 0
    lax.fori_loop(0, CHUNK // 16, fill_ones, 0)

    @pl.when(s == 0)
    def _():
        _zero_1d(zbuf, N_NODES)
        pltpu.sync_copy(zbuf, dego)
        pltpu.sync_copy(zbuf, degi)

    plsc.subcore_barrier()

    def body(j, _):
        pltpu.sync_copy(ones_v, dego.at[idx_s.at[j]], add=True)
        pltpu.sync_copy(ones_v, degi.at[idx_d.at[j]], add=True)
        return 0
    lax.fori_loop(0, NCH, body, 0)

    plsc.subcore_barrier()

    @pl.when(s == 0)
    def _():
        pltpu.sync_copy(dego, zbuf)

        @pl.when(c == 0)
        def _():
            pltpu.sync_copy(zbuf, oo0)

        @pl.when(c == 1)
        def _():
            pltpu.sync_copy(zbuf, oo1)

    @pl.when(s == 1)
    def _():
        pltpu.sync_copy(degi, zbuf)

        @pl.when(c == 0)
        def _():
            pltpu.sync_copy(zbuf, oi0)

        @pl.when(c == 1)
        def _():
            pltpu.sync_copy(zbuf, oi1)


def _agg_body(xs_hbm, src_hbm, dst_hbm, out_hbm, idx_s, idx_d, rows_v, wv, acc):
    c = lax.axis_index("c")
    s = lax.axis_index("s")
    wid = c * NS + s
    _zero_2d(wv, WCH, D)
    for t in range(WPT):
        i = s + NS * t

        @pl.when(i < NWC)
        def _():
            pltpu.sync_copy(wv, acc.at[pl.ds(i * WCH, WCH)])

    plsc.subcore_barrier()

    def group(g, _):
        pltpu.sync_copy(src_hbm.at[wid, g], idx_s)
        pltpu.sync_copy(dst_hbm.at[wid, g], idx_d)

        def body(j, _):
            pltpu.sync_copy(xs_hbm.at[idx_s.at[j]], rows_v)
            pltpu.sync_copy(rows_v, acc.at[idx_d.at[j]], add=True)
            return 0
        lax.fori_loop(0, GC, body, 0)
        return 0
    lax.fori_loop(0, NG, group, 0)

    plsc.subcore_barrier()
    for t in range(WPT):
        i = s + NS * t

        @pl.when(i < NWC)
        def _():
            pltpu.sync_copy(acc.at[pl.ds(i * WCH, WCH)], wv)
            pltpu.sync_copy(wv, out_hbm.at[c, pl.ds(i * WCH, WCH)])


_deg_call = pl.kernel(
    _deg_body,
    out_type=tuple(jax.ShapeDtypeStruct((N_NODES,), jnp.float32)
                   for _ in range(4)),
    mesh=_mesh,
    scratch_types=[
        pltpu.VMEM((NCH, CHUNK), jnp.int32),
        pltpu.VMEM((NCH, CHUNK), jnp.int32),
        pltpu.VMEM((CHUNK,), jnp.float32),
        pltpu.VMEM((N_NODES,), jnp.float32),
        pltpu.VMEM_SHARED((N_NODES,), jnp.float32),
        pltpu.VMEM_SHARED((N_NODES,), jnp.float32),
    ],
)

_agg_call = pl.kernel(
    _agg_body,
    out_type=jax.ShapeDtypeStruct((NC, N_NODES, D), jnp.float32),
    mesh=_mesh,
    scratch_types=[
        pltpu.VMEM((GC, CHUNK), jnp.int32),
        pltpu.VMEM((GC, CHUNK), jnp.int32),
        pltpu.VMEM((CHUNK, D), jnp.float32),
        pltpu.VMEM((WCH, D), jnp.float32),
        pltpu.VMEM_SHARED((N_NODES, D), jnp.float32),
    ],
)


BLK = 1000
NBLK = N_NODES // BLK


def _norm(deg):
    return jnp.where(deg > 0.0, lax.rsqrt(jnp.maximum(deg, 1.0)), 0.0)


def _prescale_body(h_ref, do0_ref, do1_ref, out_ref):
    dego = do0_ref[...] + do1_ref[...]
    ns = _norm(dego[0, 0])
    out_ref[...] = h_ref[...] * ns[:, None]


def _post_body(part_ref, do0_ref, do1_ref, di0_ref, di1_ref, w_ref, b_ref,
               out_ref, *, apply_src):
    p = part_ref[...]
    agg = p[0] + p[1]
    degi = di0_ref[...] + di1_ref[...]
    nd = _norm(degi[0, 0])[:, None]
    y = jnp.dot(agg * nd, w_ref[...],
                preferred_element_type=jnp.float32,
                precision=lax.Precision.HIGHEST) + b_ref[...]
    y = jnp.maximum(y, 0.0)
    if apply_src:
        dego = do0_ref[...] + do1_ref[...]
        y = y * _norm(dego[0, 0])[:, None]
    out_ref[...] = y


_deg_spec = pl.BlockSpec((1, 1, BLK), lambda i: (i, 0, 0))


def _prescale(h, do0, do1):
    return pl.pallas_call(
        _prescale_body,
        grid=(NBLK,),
        in_specs=[
            pl.BlockSpec((BLK, D), lambda i: (i, 0)),
            _deg_spec,
            _deg_spec,
        ],
        out_specs=pl.BlockSpec((BLK, D), lambda i: (i, 0)),
        out_shape=jax.ShapeDtypeStruct((N_NODES, D), jnp.float32),
    )(h, do0, do1)


def _post(part, do0, do1, di0, di1, w, b2d, apply_src):
    return pl.pallas_call(
        functools.partial(_post_body, apply_src=apply_src),
        grid=(NBLK,),
        in_specs=[
            pl.BlockSpec((NC, BLK, D), lambda i: (0, i, 0)),
            _deg_spec,
            _deg_spec,
            _deg_spec,
            _deg_spec,
            pl.BlockSpec((D, D), lambda i: (0, 0)),
            pl.BlockSpec((1, D), lambda i: (0, 0)),
        ],
        out_specs=pl.BlockSpec((BLK, D), lambda i: (i, 0)),
        out_shape=jax.ShapeDtypeStruct((N_NODES, D), jnp.float32),
    )(part, do0, do1, di0, di1, w, b2d)


def kernel(h, edge_index, W1, b1, W2, b2):
    ei = edge_index.astype(jnp.int32)
    src3 = ei[0].reshape(NW, NCH, CHUNK)
    dst3 = ei[1].reshape(NW, NCH, CHUNK)
    src4 = ei[0].reshape(NW, NG, GC, CHUNK)
    dst4 = ei[1].reshape(NW, NG, GC, CHUNK)
    dego0, degi0, dego1, degi1 = _deg_call(src3, dst3)
    r = lambda a: a.reshape(NBLK, 1, BLK)
    do0, do1, di0, di1 = r(dego0), r(dego1), r(degi0), r(degi1)
    xs1 = _prescale(h, do0, do1)
    part1 = _agg_call(xs1, src4, dst4)
    xs2 = _post(part1, do0, do1, di0, di1, W1, b1.reshape(1, D), True)
    part2 = _agg_call(xs2, src4, dst4)
    out = _post(part2, do0, do1, di0, di1, W2, b2.reshape(1, D), False)
    return out


# speedup vs baseline: 9.2578x; 1.2290x over previous
import functools

import jax
import jax.numpy as jnp
from jax import lax
from jax.experimental import pallas as pl
from jax.experimental.pallas import tpu as pltpu
from jax.experimental.pallas import tpu_sc as plsc

N_NODES = 10000
N_EDGES = 320000
D = 128

NC = 2
NS = 16
NW = NC * NS
EPT = N_EDGES // NW
CHUNK = 80
NCH = EPT // CHUNK
GC = 25
NG = NCH // GC
WCH = 80
NWC = N_NODES // WCH
WPT = -(-NWC // NS)

_mesh = plsc.VectorSubcoreMesh(
    core_axis_name="c", subcore_axis_name="s", num_cores=NC, num_subcores=NS)


def _zero_1d(ref, n):
    def body(i, _):
        ref[pl.ds(i * 16, 16)] = jnp.zeros((16,), jnp.float32)
        return 0
    lax.fori_loop(0, n // 16, body, 0)


def _zero_2d(ref, rows, cols):
    nlc = cols // 16
    def body(i, _):
        ref[i // nlc, pl.ds((i % nlc) * 16, 16)] = jnp.zeros((16,), jnp.float32)
        return 0
    lax.fori_loop(0, rows * nlc, body, 0)


def _deg_body(src_hbm, dst_hbm, oo0, oi0, oo1, oi1,
              idx_s, idx_d, ones_v, zbuf, dego, degi):
    c = lax.axis_index("c")
    s = lax.axis_index("s")
    wid = c * NS + s
    pltpu.sync_copy(src_hbm.at[wid], idx_s)
    pltpu.sync_copy(dst_hbm.at[wid], idx_d)

    def fill_ones(i, _):
        ones_v[pl.ds(i * 16, 16)] = jnp.ones((16,), jnp.float32)
        return 0
    lax.fori_loop(0, CHUNK // 16, fill_ones, 0)

    @pl.when(s == 0)
    def _():
        _zero_1d(zbuf, N_NODES)
        pltpu.sync_copy(zbuf, dego)
        pltpu.sync_copy(zbuf, degi)

    plsc.subcore_barrier()

    def body(j, _):
        pltpu.sync_copy(ones_v, dego.at[idx_s.at[j]], add=True)
        pltpu.sync_copy(ones_v, degi.at[idx_d.at[j]], add=True)
        return 0
    lax.fori_loop(0, NCH, body, 0)

    plsc.subcore_barrier()

    @pl.when(s == 0)
    def _():
        pltpu.sync_copy(dego, zbuf)

        @pl.when(c == 0)
        def _():
            pltpu.sync_copy(zbuf, oo0)

        @pl.when(c == 1)
        def _():
            pltpu.sync_copy(zbuf, oo1)

    @pl.when(s == 1)
    def _():
        pltpu.sync_copy(degi, zbuf)

        @pl.when(c == 0)
        def _():
            pltpu.sync_copy(zbuf, oi0)

        @pl.when(c == 1)
        def _():
            pltpu.sync_copy(zbuf, oi1)


def _agg_body(xs_hbm, src_hbm, dst_hbm, out_hbm,
              idx_s, idx_d, rows_a, rows_b, wv, acc, sem_a, sem_b):
    c = lax.axis_index("c")
    s = lax.axis_index("s")
    wid = c * NS + s
    _zero_2d(wv, WCH, D)
    for t in range(WPT):
        i = s + NS * t

        @pl.when(i < NWC)
        def _():
            pltpu.sync_copy(wv, acc.at[pl.ds(i * WCH, WCH)])

    plsc.subcore_barrier()

    def group(g, _):
        pltpu.sync_copy(src_hbm.at[wid, g], idx_s)
        pltpu.sync_copy(dst_hbm.at[wid, g], idx_d)
        pltpu.async_copy(xs_hbm.at[idx_s.at[0]], rows_a, sem_a)

        def pair(p, _):
            j = 2 * p
            pltpu.make_async_copy(xs_hbm.at[idx_s.at[j]], rows_a, sem_a).wait()
            pltpu.async_copy(xs_hbm.at[idx_s.at[j + 1]], rows_b, sem_b)
            pltpu.sync_copy(rows_a, acc.at[idx_d.at[j]], add=True)
            pltpu.make_async_copy(
                xs_hbm.at[idx_s.at[j + 1]], rows_b, sem_b).wait()
            pltpu.async_copy(xs_hbm.at[idx_s.at[j + 2]], rows_a, sem_a)
            pltpu.sync_copy(rows_b, acc.at[idx_d.at[j + 1]], add=True)
            return 0
        lax.fori_loop(0, (GC - 1) // 2, pair, 0)
        pltpu.make_async_copy(
            xs_hbm.at[idx_s.at[GC - 1]], rows_a, sem_a).wait()
        pltpu.sync_copy(rows_a, acc.at[idx_d.at[GC - 1]], add=True)
        return 0
    lax.fori_loop(0, NG, group, 0)

    plsc.subcore_barrier()
    for t in range(WPT):
        i = s + NS * t

        @pl.when(i < NWC)
        def _():
            pltpu.sync_copy(acc.at[pl.ds(i * WCH, WCH)], wv)
            pltpu.sync_copy(wv, out_hbm.at[c, pl.ds(i * WCH, WCH)])


_deg_call = pl.kernel(
    _deg_body,
    out_type=tuple(jax.ShapeDtypeStruct((N_NODES,), jnp.float32)
                   for _ in range(4)),
    mesh=_mesh,
    scratch_types=[
        pltpu.VMEM((NCH, CHUNK), jnp.int32),
        pltpu.VMEM((NCH, CHUNK), jnp.int32),
        pltpu.VMEM((CHUNK,), jnp.float32),
        pltpu.VMEM((N_NODES,), jnp.float32),
        pltpu.VMEM_SHARED((N_NODES,), jnp.float32),
        pltpu.VMEM_SHARED((N_NODES,), jnp.float32),
    ],
)

_agg_call = pl.kernel(
    _agg_body,
    out_type=jax.ShapeDtypeStruct((NC, N_NODES, D), jnp.float32),
    mesh=_mesh,
    scratch_types=[
        pltpu.VMEM((GC, CHUNK), jnp.int32),
        pltpu.VMEM((GC, CHUNK), jnp.int32),
        pltpu.VMEM((CHUNK, D), jnp.float32),
        pltpu.VMEM((CHUNK, D), jnp.float32),
        pltpu.VMEM((WCH, D), jnp.float32),
        pltpu.VMEM_SHARED((N_NODES, D), jnp.float32),
        pltpu.SemaphoreType.DMA,
        pltpu.SemaphoreType.DMA,
    ],
)


BLK = 1000
NBLK = N_NODES // BLK


def _norm(deg):
    return jnp.where(deg > 0.0, lax.rsqrt(jnp.maximum(deg, 1.0)), 0.0)


def _prescale_body(h_ref, do0_ref, do1_ref, out_ref):
    dego = do0_ref[...] + do1_ref[...]
    ns = _norm(dego[0, 0])
    out_ref[...] = h_ref[...] * ns[:, None]


def _post_body(part_ref, do0_ref, do1_ref, di0_ref, di1_ref, w_ref, b_ref,
               out_ref, *, apply_src):
    p = part_ref[...]
    agg = p[0] + p[1]
    degi = di0_ref[...] + di1_ref[...]
    nd = _norm(degi[0, 0])[:, None]
    y = jnp.dot(agg * nd, w_ref[...],
                preferred_element_type=jnp.float32,
                precision=lax.Precision.HIGHEST) + b_ref[...]
    y = jnp.maximum(y, 0.0)
    if apply_src:
        dego = do0_ref[...] + do1_ref[...]
        y = y * _norm(dego[0, 0])[:, None]
    out_ref[...] = y


_deg_spec = pl.BlockSpec((1, 1, BLK), lambda i: (i, 0, 0))


def _prescale(h, do0, do1):
    return pl.pallas_call(
        _prescale_body,
        grid=(NBLK,),
        in_specs=[
            pl.BlockSpec((BLK, D), lambda i: (i, 0)),
            _deg_spec,
            _deg_spec,
        ],
        out_specs=pl.BlockSpec((BLK, D), lambda i: (i, 0)),
        out_shape=jax.ShapeDtypeStruct((N_NODES, D), jnp.float32),
    )(h, do0, do1)


def _post(part, do0, do1, di0, di1, w, b2d, apply_src):
    return pl.pallas_call(
        functools.partial(_post_body, apply_src=apply_src),
        grid=(NBLK,),
        in_specs=[
            pl.BlockSpec((NC, BLK, D), lambda i: (0, i, 0)),
            _deg_spec,
            _deg_spec,
            _deg_spec,
            _deg_spec,
            pl.BlockSpec((D, D), lambda i: (0, 0)),
            pl.BlockSpec((1, D), lambda i: (0, 0)),
        ],
        out_specs=pl.BlockSpec((BLK, D), lambda i: (i, 0)),
        out_shape=jax.ShapeDtypeStruct((N_NODES, D), jnp.float32),
    )(part, do0, do1, di0, di1, w, b2d)


def kernel(h, edge_index, W1, b1, W2, b2):
    ei = edge_index.astype(jnp.int32)
    src3 = ei[0].reshape(NW, NCH, CHUNK)
    dst3 = ei[1].reshape(NW, NCH, CHUNK)
    src4 = ei[0].reshape(NW, NG, GC, CHUNK)
    dst4 = ei[1].reshape(NW, NG, GC, CHUNK)
    dego0, degi0, dego1, degi1 = _deg_call(src3, dst3)
    r = lambda a: a.reshape(NBLK, 1, BLK)
    do0, do1, di0, di1 = r(dego0), r(dego1), r(degi0), r(degi1)
    xs1 = _prescale(h, do0, do1)
    part1 = _agg_call(xs1, src4, dst4)
    xs2 = _post(part1, do0, do1, di0, di1, W1, b1.reshape(1, D), True)
    part2 = _agg_call(xs2, src4, dst4)
    out = _post(part2, do0, do1, di0, di1, W2, b2.reshape(1, D), False)
    return out


# speedup vs baseline: 10.6606x; 1.1515x over previous
import functools

import jax
import jax.numpy as jnp
from jax import lax
from jax.experimental import pallas as pl
from jax.experimental.pallas import tpu as pltpu
from jax.experimental.pallas import tpu_sc as plsc

N_NODES = 10000
N_EDGES = 320000
D = 128

NC = 2
NS = 16
NW = NC * NS
EPT = N_EDGES // NW
CHUNK = 125
NCH = EPT // CHUNK
GC = 16
NG = NCH // GC
WCH = 80
NWC = N_NODES // WCH
WPT = -(-NWC // NS)

_mesh = plsc.VectorSubcoreMesh(
    core_axis_name="c", subcore_axis_name="s", num_cores=NC, num_subcores=NS)


def _zero_1d(ref, n):
    def body(i, _):
        ref[pl.ds(i * 16, 16)] = jnp.zeros((16,), jnp.float32)
        return 0
    lax.fori_loop(0, n // 16, body, 0)


def _zero_2d(ref, rows, cols):
    nlc = cols // 16
    def body(i, _):
        ref[i // nlc, pl.ds((i % nlc) * 16, 16)] = jnp.zeros((16,), jnp.float32)
        return 0
    lax.fori_loop(0, rows * nlc, body, 0)


def _deg_body(src_hbm, dst_hbm, oo0, oi0, oo1, oi1,
              idx_s, idx_d, ones_v, zbuf, dego, degi):
    c = lax.axis_index("c")
    s = lax.axis_index("s")
    wid = c * NS + s
    pltpu.sync_copy(src_hbm.at[wid], idx_s)
    pltpu.sync_copy(dst_hbm.at[wid], idx_d)

    def fill_ones(i, _):
        ones_v[pl.ds(i * 16, 16)] = jnp.ones((16,), jnp.float32)
        return 0
    lax.fori_loop(0, CHUNK // 16, fill_ones, 0)
    if CHUNK % 16:
        ones_v[pl.ds(CHUNK - 16, 16)] = jnp.ones((16,), jnp.float32)

    @pl.when(s == 0)
    def _():
        _zero_1d(zbuf, N_NODES)
        pltpu.sync_copy(zbuf, dego)
        pltpu.sync_copy(zbuf, degi)

    plsc.subcore_barrier()

    def body(j, _):
        pltpu.sync_copy(ones_v, dego.at[idx_s.at[j]], add=True)
        pltpu.sync_copy(ones_v, degi.at[idx_d.at[j]], add=True)
        return 0
    lax.fori_loop(0, NCH, body, 0)

    plsc.subcore_barrier()

    @pl.when(s == 0)
    def _():
        pltpu.sync_copy(dego, zbuf)

        @pl.when(c == 0)
        def _():
            pltpu.sync_copy(zbuf, oo0)

        @pl.when(c == 1)
        def _():
            pltpu.sync_copy(zbuf, oo1)

    @pl.when(s == 1)
    def _():
        pltpu.sync_copy(degi, zbuf)

        @pl.when(c == 0)
        def _():
            pltpu.sync_copy(zbuf, oi0)

        @pl.when(c == 1)
        def _():
            pltpu.sync_copy(zbuf, oi1)


def _agg_body(xs_hbm, src_hbm, dst_hbm, out_hbm,
              idx_s, idx_d, rows_a, rows_b, wv, acc, sem_a, sem_b):
    c = lax.axis_index("c")
    s = lax.axis_index("s")
    wid = c * NS + s
    _zero_2d(wv, WCH, D)
    for t in range(WPT):
        i = s + NS * t

        @pl.when(i < NWC)
        def _():
            pltpu.sync_copy(wv, acc.at[pl.ds(i * WCH, WCH)])

    plsc.subcore_barrier()

    def group(g, _):
        pltpu.sync_copy(src_hbm.at[wid, g], idx_s)
        pltpu.sync_copy(dst_hbm.at[wid, g], idx_d)
        pltpu.async_copy(xs_hbm.at[idx_s.at[0]], rows_a, sem_a)

        def pair(p, _):
            j = 2 * p
            pltpu.make_async_copy(xs_hbm.at[idx_s.at[j]], rows_a, sem_a).wait()
            pltpu.async_copy(xs_hbm.at[idx_s.at[j + 1]], rows_b, sem_b)
            pltpu.sync_copy(rows_a, acc.at[idx_d.at[j]], add=True)
            pltpu.make_async_copy(
                xs_hbm.at[idx_s.at[j + 1]], rows_b, sem_b).wait()
            pltpu.async_copy(xs_hbm.at[idx_s.at[j + 2]], rows_a, sem_a)
            pltpu.sync_copy(rows_b, acc.at[idx_d.at[j + 1]], add=True)
            return 0
        lax.fori_loop(0, (GC - 1) // 2, pair, 0)
        if GC % 2 == 0:
            pltpu.make_async_copy(
                xs_hbm.at[idx_s.at[GC - 2]], rows_a, sem_a).wait()
            pltpu.async_copy(xs_hbm.at[idx_s.at[GC - 1]], rows_b, sem_b)
            pltpu.sync_copy(rows_a, acc.at[idx_d.at[GC - 2]], add=True)
            pltpu.make_async_copy(
                xs_hbm.at[idx_s.at[GC - 1]], rows_b, sem_b).wait()
            pltpu.sync_copy(rows_b, acc.at[idx_d.at[GC - 1]], add=True)
        else:
            pltpu.make_async_copy(
                xs_hbm.at[idx_s.at[GC - 1]], rows_a, sem_a).wait()
            pltpu.sync_copy(rows_a, acc.at[idx_d.at[GC - 1]], add=True)
        return 0
    lax.fori_loop(0, NG, group, 0)

    plsc.subcore_barrier()
    for t in range(WPT):
        i = s + NS * t

        @pl.when(i < NWC)
        def _():
            pltpu.sync_copy(acc.at[pl.ds(i * WCH, WCH)], wv)
            pltpu.sync_copy(wv, out_hbm.at[c, pl.ds(i * WCH, WCH)])


_deg_call = pl.kernel(
    _deg_body,
    out_type=tuple(jax.ShapeDtypeStruct((N_NODES,), jnp.float32)
                   for _ in range(4)),
    mesh=_mesh,
    scratch_types=[
        pltpu.VMEM((NCH, CHUNK), jnp.int32),
        pltpu.VMEM((NCH, CHUNK), jnp.int32),
        pltpu.VMEM((CHUNK,), jnp.float32),
        pltpu.VMEM((N_NODES,), jnp.float32),
        pltpu.VMEM_SHARED((N_NODES,), jnp.float32),
        pltpu.VMEM_SHARED((N_NODES,), jnp.float32),
    ],
)

_agg_call = pl.kernel(
    _agg_body,
    out_type=jax.ShapeDtypeStruct((NC, N_NODES, D), jnp.float32),
    mesh=_mesh,
    scratch_types=[
        pltpu.VMEM((GC, CHUNK), jnp.int32),
        pltpu.VMEM((GC, CHUNK), jnp.int32),
        pltpu.VMEM((CHUNK, D), jnp.float32),
        pltpu.VMEM((CHUNK, D), jnp.float32),
        pltpu.VMEM((WCH, D), jnp.float32),
        pltpu.VMEM_SHARED((N_NODES, D), jnp.float32),
        pltpu.SemaphoreType.DMA,
        pltpu.SemaphoreType.DMA,
    ],
)


BLK = 1000
NBLK = N_NODES // BLK


def _norm(deg):
    return jnp.where(deg > 0.0, lax.rsqrt(jnp.maximum(deg, 1.0)), 0.0)


def _prescale_body(h_ref, do0_ref, do1_ref, out_ref):
    dego = do0_ref[...] + do1_ref[...]
    ns = _norm(dego[0, 0])
    out_ref[...] = h_ref[...] * ns[:, None]


def _post_body(part_ref, do0_ref, do1_ref, di0_ref, di1_ref, w_ref, b_ref,
               out_ref, *, apply_src):
    p = part_ref[...]
    agg = p[0] + p[1]
    degi = di0_ref[...] + di1_ref[...]
    nd = _norm(degi[0, 0])[:, None]
    y = jnp.dot(agg * nd, w_ref[...],
                preferred_element_type=jnp.float32,
                precision=lax.Precision.HIGHEST) + b_ref[...]
    y = jnp.maximum(y, 0.0)
    if apply_src:
        dego = do0_ref[...] + do1_ref[...]
        y = y * _norm(dego[0, 0])[:, None]
    out_ref[...] = y


_deg_spec = pl.BlockSpec((1, 1, BLK), lambda i: (i, 0, 0))


def _prescale(h, do0, do1):
    return pl.pallas_call(
        _prescale_body,
        grid=(NBLK,),
        in_specs=[
            pl.BlockSpec((BLK, D), lambda i: (i, 0)),
            _deg_spec,
            _deg_spec,
        ],
        out_specs=pl.BlockSpec((BLK, D), lambda i: (i, 0)),
        out_shape=jax.ShapeDtypeStruct((N_NODES, D), jnp.float32),
    )(h, do0, do1)


def _post(part, do0, do1, di0, di1, w, b2d, apply_src):
    return pl.pallas_call(
        functools.partial(_post_body, apply_src=apply_src),
        grid=(NBLK,),
        in_specs=[
            pl.BlockSpec((NC, BLK, D), lambda i: (0, i, 0)),
            _deg_spec,
            _deg_spec,
            _deg_spec,
            _deg_spec,
            pl.BlockSpec((D, D), lambda i: (0, 0)),
            pl.BlockSpec((1, D), lambda i: (0, 0)),
        ],
        out_specs=pl.BlockSpec((BLK, D), lambda i: (i, 0)),
        out_shape=jax.ShapeDtypeStruct((N_NODES, D), jnp.float32),
    )(part, do0, do1, di0, di1, w, b2d)


def kernel(h, edge_index, W1, b1, W2, b2):
    ei = edge_index.astype(jnp.int32)
    src3 = ei[0].reshape(NW, NCH, CHUNK)
    dst3 = ei[1].reshape(NW, NCH, CHUNK)
    src4 = ei[0].reshape(NW, NG, GC, CHUNK)
    dst4 = ei[1].reshape(NW, NG, GC, CHUNK)
    dego0, degi0, dego1, degi1 = _deg_call(src3, dst3)
    r = lambda a: a.reshape(NBLK, 1, BLK)
    do0, do1, di0, di1 = r(dego0), r(dego1), r(degi0), r(degi1)
    xs1 = _prescale(h, do0, do1)
    part1 = _agg_call(xs1, src4, dst4)
    xs2 = _post(part1, do0, do1, di0, di1, W1, b1.reshape(1, D), True)
    part2 = _agg_call(xs2, src4, dst4)
    out = _post(part2, do0, do1, di0, di1, W2, b2.reshape(1, D), False)
    return out
